# baseline (speedup 1.0000x reference)
import functools

import jax
import jax.numpy as jnp
from jax import lax
from jax.experimental import pallas as pl
from jax.experimental.pallas import tpu as pltpu
from jax.experimental.pallas import tpu_sc as plsc

_NC = 2
_CHUNK_ROWS = 256
_NBUF = 6


def kernel(x, pos_emb):
    T = x.shape[1]
    D = pos_emb.shape[1]
    rows_per_c = T // _NC
    n_chunks = rows_per_c // _CHUNK_ROWS
    mesh = plsc.ScalarSubcoreMesh(axis_name="c", num_cores=_NC)

    @functools.partial(
        pl.kernel,
        mesh=mesh,
        out_type=jax.ShapeDtypeStruct((T, D), pos_emb.dtype),
        scratch_types=[
            pltpu.VMEM_SHARED((_NBUF, _CHUNK_ROWS, D), pos_emb.dtype),
            pltpu.SemaphoreType.DMA((_NBUF,)),
            pltpu.SemaphoreType.DMA((_NBUF,)),
        ],
    )
    def sc_copy(emb_hbm, out_hbm, bufs, sem_in, sem_out):
        base = lax.axis_index("c") * rows_per_c

        def start_in(j):
            r = base + j * _CHUNK_ROWS
            return pltpu.async_copy(
                emb_hbm.at[pl.ds(r, _CHUNK_ROWS), :],
                bufs.at[j % _NBUF],
                sem_in.at[j % _NBUF],
            )

        def start_out(j):
            r = base + j * _CHUNK_ROWS
            return pltpu.async_copy(
                bufs.at[j % _NBUF],
                out_hbm.at[pl.ds(r, _CHUNK_ROWS), :],
                sem_out.at[j % _NBUF],
            )

        ins = [None] * n_chunks
        outs = [None] * n_chunks
        for j in range(min(_NBUF, n_chunks)):
            ins[j] = start_in(j)
        for j in range(n_chunks):
            ins[j].wait()
            outs[j] = start_out(j)
            nxt = j + _NBUF
            if nxt < n_chunks:
                outs[j].wait()
                ins[nxt] = start_in(nxt)
        for j in range(max(0, n_chunks - _NBUF), n_chunks):
            outs[j].wait()

    return sc_copy(pos_emb).reshape(1, T, D)


# speedup vs baseline: 2.6999x; 2.6999x over previous
import jax
import jax.numpy as jnp
from jax.experimental import pallas as pl
from jax.experimental.pallas import tpu as pltpu

_CHUNKS = (256, 512, 768, 1024, 1536)


def _dma_copy(emb_ref, out_ref, scratch, sems):
    n = len(_CHUNKS)
    offs = [sum(_CHUNKS[:i]) for i in range(n)]

    def in_copy(i):
        return pltpu.make_async_copy(
            emb_ref.at[pl.ds(offs[i], _CHUNKS[i]), :],
            scratch.at[pl.ds(offs[i], _CHUNKS[i]), :],
            sems.at[i],
        )

    def out_copy(i):
        return pltpu.make_async_copy(
            scratch.at[pl.ds(offs[i], _CHUNKS[i]), :],
            out_ref.at[0, pl.ds(offs[i], _CHUNKS[i]), :],
            sems.at[n + i],
        )

    for i in range(n):
        in_copy(i).start()
    for i in range(n):
        in_copy(i).wait()
        out_copy(i).start()
    for i in range(n):
        out_copy(i).wait()


def kernel(x, pos_emb):
    T = x.shape[1]
    D = pos_emb.shape[1]
    n = len(_CHUNKS)
    out = pl.pallas_call(
        _dma_copy,
        in_specs=[pl.BlockSpec(memory_space=pltpu.MemorySpace.HBM)],
        out_specs=pl.BlockSpec(memory_space=pltpu.MemorySpace.HBM),
        out_shape=jax.ShapeDtypeStruct((1, T, D), pos_emb.dtype),
        scratch_shapes=[
            pltpu.VMEM((T, D), pos_emb.dtype),
            pltpu.SemaphoreType.DMA((2 * n,)),
        ],
    )(pos_emb)
    return out


# speedup vs baseline: 2.8541x; 1.0571x over previous
import jax
import jax.numpy as jnp
from jax.experimental import pallas as pl


def _copy_block(emb_ref, out_ref):
    out_ref[0, :, :] = emb_ref[:, :]


def kernel(x, pos_emb):
    T = x.shape[1]
    D = pos_emb.shape[1]
    R = 2048
    out = pl.pallas_call(
        _copy_block,
        grid=(T // R,),
        in_specs=[pl.BlockSpec((R, D), lambda i: (i, 0))],
        out_specs=pl.BlockSpec((1, R, D), lambda i: (0, i, 0)),
        out_shape=jax.ShapeDtypeStruct((1, T, D), pos_emb.dtype),
    )(pos_emb)
    return out
